# baseline (speedup 1.0000x reference)
import functools

import jax
import jax.numpy as jnp
from jax.experimental import pallas as pl
from jax.experimental.pallas import tpu as pltpu

_K = 3
_LT = 128


def _body(csum_ref, halo_ref, x_ref, o_ref, mp_ref, mn_ref, *, rows_per_blk):
    i = pl.program_id(0)
    a = pl.program_id(1)
    xb = x_ref[...]
    R = xb.shape[0]

    @pl.when(a == 0)
    def _():
        g = i * rows_per_blk + jax.lax.broadcasted_iota(jnp.int32, (R, 1), 0)
        csum = csum_ref[...]
        hit = ((g == csum).astype(jnp.int32)
               + 2 * ((g + 1) == csum).astype(jnp.int32))
        red = jnp.max(hit, axis=1, keepdims=True)
        is_start = ((red & 1) > 0) | (g == 0)
        is_end = (red & 2) > 0
        mp_ref[...] = jnp.where(is_start, 0.0, 1.0)
        mn_ref[...] = jnp.where(is_end, 0.0, 1.0)

    prev_m = jnp.concatenate([halo_ref[0, 0:1, :], xb[:-1, :]], axis=0) * mp_ref[...]
    next_m = jnp.concatenate([xb[1:, :], halo_ref[0, 1:2, :]], axis=0) * mn_ref[...]

    mod = jax.lax.broadcasted_iota(jnp.int32, (R, _K * _LT), 1) % _K
    idx = jax.lax.broadcasted_iota(jnp.int32, (R, _K * _LT), 1) // _K
    pa = jnp.take_along_axis(prev_m, idx, axis=1)
    ca = jnp.take_along_axis(xb, idx, axis=1)
    na = jnp.take_along_axis(next_m, idx, axis=1)
    o_ref[...] = jnp.where(mod == 0, pa, jnp.where(mod == 1, ca, na))


def kernel(x, lengths):
    N, d = x.shape
    lens = lengths.astype(jnp.int32)
    csum = jnp.cumsum(lens).reshape(1, -1)

    R = 640
    assert N % R == 0 and d % _LT == 0
    nblk = N // R
    na = d // _LT

    blk = jnp.arange(nblk, dtype=jnp.int32)
    prev_idx = jnp.maximum(blk * R - 1, 0)
    next_idx = jnp.minimum((blk + 1) * R, N - 1)
    halo = jnp.stack([x[prev_idx], x[next_idx]], axis=1)

    out = pl.pallas_call(
        functools.partial(_body, rows_per_blk=R),
        grid=(nblk, na),
        in_specs=[
            pl.BlockSpec((1, csum.shape[1]), lambda i, a: (0, 0)),
            pl.BlockSpec((1, 2, _LT), lambda i, a: (i, 0, a)),
            pl.BlockSpec((R, _LT), lambda i, a: (i, a)),
        ],
        out_specs=pl.BlockSpec((R, _K * _LT), lambda i, a: (i, a)),
        out_shape=jax.ShapeDtypeStruct((N, d * _K), x.dtype),
        scratch_shapes=[
            pltpu.VMEM((R, 1), jnp.float32),
            pltpu.VMEM((R, 1), jnp.float32),
        ],
    )(csum, halo, x)
    return out
